# baseline (speedup 1.0000x reference)
import jax
import jax.numpy as jnp
from jax.experimental import pallas as pl
from jax.experimental.pallas import tpu as pltpu

N_TOKENS = 8192
HIDDEN = 2048
N_EXPERTS = 16
TOPK = 2
AUX_COEF = 0.001
Z_COEF = 0.001
BLK = 1024
NBLK = N_TOKENS // BLK


def _router_kernel(x_ref, wt_ref, w_out, i_out, cnt_out, psum_out, z_out):
    logits = jnp.dot(x_ref[...], wt_ref[...],
                     preferred_element_type=jnp.float32)
    iota = jax.lax.broadcasted_iota(
        jnp.int32, logits.shape, 1).astype(jnp.float32)

    m1 = jnp.max(logits, axis=1, keepdims=True)
    i1 = jnp.min(jnp.where(logits == m1, iota, float(N_EXPERTS)),
                 axis=1, keepdims=True)
    sel1 = iota == i1
    masked = jnp.where(sel1, -jnp.inf, logits)
    m2 = jnp.max(masked, axis=1, keepdims=True)
    i2 = jnp.min(jnp.where(masked == m2, iota, float(N_EXPERTS)),
                 axis=1, keepdims=True)
    sel2 = iota == i2

    e2 = jnp.exp(m2 - m1)
    w2 = e2 / (1.0 + e2)
    w_out[...] = jnp.concatenate([1.0 - w2, w2], axis=1)
    i_out[...] = jnp.concatenate([i1, i2], axis=1).astype(jnp.int32)

    ex = jnp.exp(logits - m1)
    denom = jnp.sum(ex, axis=1, keepdims=True)
    probs = ex * (1.0 / denom)
    ones_row = jnp.ones((1, probs.shape[0]), dtype=jnp.float32)
    contrib = jnp.where(sel1, 1.0, 0.0) + jnp.where(sel2, 1.0, 0.0)
    cnt_out[0, :, :] = jnp.dot(ones_row, contrib,
                               preferred_element_type=jnp.float32)
    psum_out[0, :, :] = jnp.dot(ones_row, probs,
                                preferred_element_type=jnp.float32)
    log_z = m1 + jnp.log(denom)
    z_out[0, :, :] = jnp.dot(ones_row, log_z * log_z,
                             preferred_element_type=jnp.float32)


def _combine_kernel(cnt_ref, psum_ref, z_ref, aux_out):
    f = jnp.sum(cnt_ref[...], axis=0) / (N_TOKENS * TOPK)
    p_mean = jnp.sum(psum_ref[...], axis=0) / N_TOKENS
    lb_loss = N_EXPERTS * jnp.sum(f * p_mean)
    z_loss = jnp.sum(z_ref[...]) / N_TOKENS
    aux_out[0, 0] = AUX_COEF * lb_loss + Z_COEF * z_loss


@jax.jit
def kernel(hidden_states, gate_weight):
    wt = gate_weight.T
    grid = (NBLK,)
    weights, indices, cnt_p, psum_p, z_p = pl.pallas_call(
        _router_kernel,
        grid=grid,
        in_specs=[
            pl.BlockSpec((BLK, HIDDEN), lambda i: (i, 0)),
            pl.BlockSpec((HIDDEN, N_EXPERTS), lambda i: (0, 0)),
        ],
        out_specs=[
            pl.BlockSpec((BLK, TOPK), lambda i: (i, 0)),
            pl.BlockSpec((BLK, TOPK), lambda i: (i, 0)),
            pl.BlockSpec((1, 1, N_EXPERTS), lambda i: (i, 0, 0)),
            pl.BlockSpec((1, 1, N_EXPERTS), lambda i: (i, 0, 0)),
            pl.BlockSpec((1, 1, 1), lambda i: (i, 0, 0)),
        ],
        out_shape=[
            jax.ShapeDtypeStruct((N_TOKENS, TOPK), jnp.float32),
            jax.ShapeDtypeStruct((N_TOKENS, TOPK), jnp.int32),
            jax.ShapeDtypeStruct((NBLK, 1, N_EXPERTS), jnp.float32),
            jax.ShapeDtypeStruct((NBLK, 1, N_EXPERTS), jnp.float32),
            jax.ShapeDtypeStruct((NBLK, 1, 1), jnp.float32),
        ],
        compiler_params=pltpu.CompilerParams(
            dimension_semantics=("parallel",)),
    )(hidden_states, wt)

    aux = pl.pallas_call(
        _combine_kernel,
        out_specs=pl.BlockSpec(memory_space=pltpu.SMEM),
        out_shape=jax.ShapeDtypeStruct((1, 1), jnp.float32),
    )(cnt_p, psum_p, z_p)
    return weights, indices, aux[0, 0]


# speedup vs baseline: 1.0198x; 1.0198x over previous
import jax
import jax.numpy as jnp
from jax.experimental import pallas as pl
from jax.experimental.pallas import tpu as pltpu

N_TOKENS = 8192
HIDDEN = 2048
N_EXPERTS = 16
TOPK = 2
AUX_COEF = 0.001
Z_COEF = 0.001
BLK = 1024


def _router_kernel(x_ref, wt_ref, w_out, i_out, aux_out,
                   cnt_ref, psum_ref, zsum_ref):
    step = pl.program_id(0)
    nsteps = pl.num_programs(0)

    @pl.when(step == 0)
    def _init():
        cnt_ref[...] = jnp.zeros_like(cnt_ref)
        psum_ref[...] = jnp.zeros_like(psum_ref)
        zsum_ref[...] = jnp.zeros_like(zsum_ref)

    logits = jnp.dot(x_ref[...], wt_ref[...],
                     preferred_element_type=jnp.float32)
    iota = jax.lax.broadcasted_iota(
        jnp.int32, logits.shape, 1).astype(jnp.float32)

    m1 = jnp.max(logits, axis=1, keepdims=True)
    i1 = jnp.min(jnp.where(logits == m1, iota, float(N_EXPERTS)),
                 axis=1, keepdims=True)
    sel1 = iota == i1
    masked = jnp.where(sel1, -jnp.inf, logits)
    m2 = jnp.max(masked, axis=1, keepdims=True)
    i2 = jnp.min(jnp.where(masked == m2, iota, float(N_EXPERTS)),
                 axis=1, keepdims=True)
    sel2 = iota == i2

    e2 = jnp.exp(m2 - m1)
    w2 = e2 / (1.0 + e2)
    w_out[...] = jnp.concatenate([1.0 - w2, w2], axis=1)
    i_out[...] = jnp.concatenate([i1, i2], axis=1).astype(jnp.int32)

    ex = jnp.exp(logits - m1)
    denom = jnp.sum(ex, axis=1, keepdims=True)
    probs = ex * (1.0 / denom)
    ones_row = jnp.ones((1, probs.shape[0]), dtype=jnp.float32)
    contrib = jnp.where(sel1, 1.0, 0.0) + jnp.where(sel2, 1.0, 0.0)
    cnt_ref[...] += jnp.dot(ones_row, contrib,
                            preferred_element_type=jnp.float32)
    psum_ref[...] += jnp.dot(ones_row, probs,
                             preferred_element_type=jnp.float32)
    log_z = m1 + jnp.log(denom)
    zsum_ref[...] += jnp.dot(ones_row, log_z * log_z,
                             preferred_element_type=jnp.float32)

    @pl.when(step == nsteps - 1)
    def _fin():
        f = cnt_ref[...] / (N_TOKENS * TOPK)
        p_mean = psum_ref[...] / N_TOKENS
        lb_loss = N_EXPERTS * jnp.sum(f * p_mean)
        z_loss = zsum_ref[0, 0] / N_TOKENS
        aux_out[0, 0] = AUX_COEF * lb_loss + Z_COEF * z_loss


@jax.jit
def kernel(hidden_states, gate_weight):
    wt = gate_weight.T
    grid = (N_TOKENS // BLK,)
    weights, indices, aux = pl.pallas_call(
        _router_kernel,
        grid=grid,
        in_specs=[
            pl.BlockSpec((BLK, HIDDEN), lambda i: (i, 0)),
            pl.BlockSpec((HIDDEN, N_EXPERTS), lambda i: (0, 0)),
        ],
        out_specs=[
            pl.BlockSpec((BLK, TOPK), lambda i: (i, 0)),
            pl.BlockSpec((BLK, TOPK), lambda i: (i, 0)),
            pl.BlockSpec(memory_space=pltpu.SMEM),
        ],
        out_shape=[
            jax.ShapeDtypeStruct((N_TOKENS, TOPK), jnp.float32),
            jax.ShapeDtypeStruct((N_TOKENS, TOPK), jnp.int32),
            jax.ShapeDtypeStruct((1, 1), jnp.float32),
        ],
        scratch_shapes=[
            pltpu.VMEM((1, N_EXPERTS), jnp.float32),
            pltpu.VMEM((1, N_EXPERTS), jnp.float32),
            pltpu.VMEM((1, 1), jnp.float32),
        ],
    )(hidden_states, wt)
    return weights, indices, aux[0, 0]


# speedup vs baseline: 1.0216x; 1.0018x over previous
import jax
import jax.numpy as jnp
from jax.experimental import pallas as pl
from jax.experimental.pallas import tpu as pltpu

N_TOKENS = 8192
HIDDEN = 2048
N_EXPERTS = 16
TOPK = 2
AUX_COEF = 0.001
Z_COEF = 0.001
CHUNK = 512
NCHUNK = N_TOKENS // CHUNK
DEPTH = 4


def _router_kernel(x_hbm, wt_ref, w_out, i_out, aux_out, x_buf, sem):
    def start_copy(c, slot):
        pltpu.make_async_copy(
            x_hbm.at[pl.ds(c * CHUNK, CHUNK), :],
            x_buf.at[slot],
            sem.at[slot],
        ).start()

    for s in range(DEPTH):
        start_copy(s, s)

    def body(c, carry):
        cnt, psum, zsum = carry
        slot = jax.lax.rem(c, DEPTH)
        pltpu.make_async_copy(
            x_hbm.at[pl.ds(c * CHUNK, CHUNK), :],
            x_buf.at[slot],
            sem.at[slot],
        ).wait()

        logits = jnp.dot(x_buf[slot], wt_ref[...],
                         preferred_element_type=jnp.float32)

        nxt = c + DEPTH

        @pl.when(nxt < NCHUNK)
        def _prefetch():
            start_copy(nxt, slot)

        iota = jax.lax.broadcasted_iota(
            jnp.int32, logits.shape, 1).astype(jnp.float32)
        m1 = jnp.max(logits, axis=1, keepdims=True)
        i1 = jnp.min(jnp.where(logits == m1, iota, float(N_EXPERTS)),
                     axis=1, keepdims=True)
        sel1 = iota == i1
        masked = jnp.where(sel1, -jnp.inf, logits)
        m2 = jnp.max(masked, axis=1, keepdims=True)
        i2 = jnp.min(jnp.where(masked == m2, iota, float(N_EXPERTS)),
                     axis=1, keepdims=True)
        sel2 = iota == i2

        e2 = jnp.exp(m2 - m1)
        w2 = e2 / (1.0 + e2)
        row = pl.ds(c * CHUNK, CHUNK)
        w_out[row, :] = jnp.concatenate([1.0 - w2, w2], axis=1)
        i_out[row, :] = jnp.concatenate([i1, i2], axis=1).astype(jnp.int32)

        ex = jnp.exp(logits - m1)
        denom = jnp.sum(ex, axis=1, keepdims=True)
        probs = ex * (1.0 / denom)
        ones_row = jnp.ones((1, CHUNK), dtype=jnp.float32)
        contrib = jnp.where(sel1, 1.0, 0.0) + jnp.where(sel2, 1.0, 0.0)
        cnt = cnt + jnp.dot(ones_row, contrib,
                            preferred_element_type=jnp.float32)
        psum = psum + jnp.dot(ones_row, probs,
                              preferred_element_type=jnp.float32)
        log_z = m1 + jnp.log(denom)
        zsum = zsum + jnp.dot(ones_row, log_z * log_z,
                              preferred_element_type=jnp.float32)
        return cnt, psum, zsum

    init = (jnp.zeros((1, N_EXPERTS), jnp.float32),
            jnp.zeros((1, N_EXPERTS), jnp.float32),
            jnp.zeros((1, 1), jnp.float32))
    cnt, psum, zsum = jax.lax.fori_loop(0, NCHUNK, body, init)

    f = cnt / (N_TOKENS * TOPK)
    p_mean = psum / N_TOKENS
    lb_loss = N_EXPERTS * jnp.sum(f * p_mean)
    z_loss = zsum[0, 0] / N_TOKENS
    aux_out[0, 0] = AUX_COEF * lb_loss + Z_COEF * z_loss


@jax.jit
def kernel(hidden_states, gate_weight):
    wt = gate_weight.T
    weights, indices, aux = pl.pallas_call(
        _router_kernel,
        in_specs=[
            pl.BlockSpec(memory_space=pl.ANY),
            pl.BlockSpec(memory_space=pltpu.VMEM),
        ],
        out_specs=[
            pl.BlockSpec(memory_space=pltpu.VMEM),
            pl.BlockSpec(memory_space=pltpu.VMEM),
            pl.BlockSpec(memory_space=pltpu.SMEM),
        ],
        out_shape=[
            jax.ShapeDtypeStruct((N_TOKENS, TOPK), jnp.float32),
            jax.ShapeDtypeStruct((N_TOKENS, TOPK), jnp.int32),
            jax.ShapeDtypeStruct((1, 1), jnp.float32),
        ],
        scratch_shapes=[
            pltpu.VMEM((DEPTH, CHUNK, HIDDEN), jnp.float32),
            pltpu.SemaphoreType.DMA((DEPTH,)),
        ],
    )(hidden_states, wt)
    return weights, indices, aux[0, 0]


# speedup vs baseline: 1.2180x; 1.1922x over previous
import jax
import jax.numpy as jnp
from jax.experimental import pallas as pl
from jax.experimental.pallas import tpu as pltpu

N_TOKENS = 8192
HIDDEN = 2048
N_EXPERTS = 16
TOPK = 2
AUX_COEF = 0.001
Z_COEF = 0.001
BLK = 1024


def _router_kernel(x_ref, w_ref, w_out, i_out, aux_out,
                   cnt_ref, psum_ref, zsum_ref):
    step = pl.program_id(0)
    nsteps = pl.num_programs(0)

    @pl.when(step == 0)
    def _init():
        cnt_ref[...] = jnp.zeros_like(cnt_ref)
        psum_ref[...] = jnp.zeros_like(psum_ref)
        zsum_ref[...] = jnp.zeros_like(zsum_ref)

    logits_t = jax.lax.dot_general(
        w_ref[...], x_ref[...], (((1,), (1,)), ((), ())),
        preferred_element_type=jnp.float32)
    iota0 = jax.lax.broadcasted_iota(
        jnp.int32, logits_t.shape, 0).astype(jnp.float32)

    m1 = jnp.max(logits_t, axis=0, keepdims=True)
    i1 = jnp.min(jnp.where(logits_t == m1, iota0, float(N_EXPERTS)),
                 axis=0, keepdims=True)
    sel1 = iota0 == i1
    masked = jnp.where(sel1, -jnp.inf, logits_t)
    m2 = jnp.max(masked, axis=0, keepdims=True)
    i2 = jnp.min(jnp.where(masked == m2, iota0, float(N_EXPERTS)),
                 axis=0, keepdims=True)
    sel2 = iota0 == i2

    e2 = jnp.exp(m2 - m1)
    w2 = e2 / (1.0 + e2)
    packed = jnp.concatenate([1.0 - w2, w2, i1, i2], axis=0)
    packed_t = packed.T
    w_out[...] = packed_t[:, :TOPK]
    i_out[...] = packed_t[:, TOPK:].astype(jnp.int32)

    ex = jnp.exp(logits_t - m1)
    denom = jnp.sum(ex, axis=0, keepdims=True)
    probs = ex * (1.0 / denom)
    ones_col = jnp.ones((probs.shape[1], 1), dtype=jnp.float32)
    contrib = jnp.where(sel1, 1.0, 0.0) + jnp.where(sel2, 1.0, 0.0)
    cnt_ref[...] += jnp.dot(contrib, ones_col,
                            preferred_element_type=jnp.float32)
    psum_ref[...] += jnp.dot(probs, ones_col,
                             preferred_element_type=jnp.float32)
    log_z = m1 + jnp.log(denom)
    zsum_ref[...] += jnp.dot(log_z * log_z, ones_col,
                             preferred_element_type=jnp.float32)

    @pl.when(step == nsteps - 1)
    def _fin():
        f = cnt_ref[...] / (N_TOKENS * TOPK)
        p_mean = psum_ref[...] / N_TOKENS
        lb_loss = N_EXPERTS * jnp.sum(f * p_mean)
        z_loss = zsum_ref[0, 0] / N_TOKENS
        aux_out[0, 0] = AUX_COEF * lb_loss + Z_COEF * z_loss


@jax.jit
def kernel(hidden_states, gate_weight):
    grid = (N_TOKENS // BLK,)
    weights, indices, aux = pl.pallas_call(
        _router_kernel,
        grid=grid,
        in_specs=[
            pl.BlockSpec((BLK, HIDDEN), lambda i: (i, 0)),
            pl.BlockSpec((N_EXPERTS, HIDDEN), lambda i: (0, 0)),
        ],
        out_specs=[
            pl.BlockSpec((BLK, TOPK), lambda i: (i, 0)),
            pl.BlockSpec((BLK, TOPK), lambda i: (i, 0)),
            pl.BlockSpec(memory_space=pltpu.SMEM),
        ],
        out_shape=[
            jax.ShapeDtypeStruct((N_TOKENS, TOPK), jnp.float32),
            jax.ShapeDtypeStruct((N_TOKENS, TOPK), jnp.int32),
            jax.ShapeDtypeStruct((1, 1), jnp.float32),
        ],
        scratch_shapes=[
            pltpu.VMEM((N_EXPERTS, 1), jnp.float32),
            pltpu.VMEM((N_EXPERTS, 1), jnp.float32),
            pltpu.VMEM((1, 1), jnp.float32),
        ],
    )(hidden_states, gate_weight)
    return weights, indices, aux[0, 0]
